# Initial kernel scaffold; baseline (speedup 1.0000x reference)
import jax
import jax.numpy as jnp
from jax.experimental import pallas as pl


def kernel(nodes, receivers, senders, Wq, bq, Wl, bl):
    raise NotImplementedError("write your pallas kernel here")


# speedup vs baseline: 43.5895x; 43.5895x over previous
import functools

import jax
import jax.numpy as jnp
from jax import lax
from jax.experimental import pallas as pl
from jax.experimental.pallas import tpu as pltpu
from jax.experimental.pallas import tpu_sc as plsc

N = 10000
E = 320000
DIM = 128
H = 8
PH = 16

CH = 128
NSUB = 16
NPT = 624
NTAIL = N - NSUB * NPT
NCN = 48
AW = 72
ROW_BLOCK = 400


def _tc_body(x_ref, wq_ref, bq_ref, sw_ref, tb_ref, nt2_ref, t_ref):
    nt = jnp.dot(x_ref[...], wq_ref[...], preferred_element_type=jnp.float32)
    nt = nt + bq_ref[...]
    lr = jnp.where(nt >= 0, nt, 0.2 * nt)
    t_ref[...] = (
        jnp.dot(lr, sw_ref[...], preferred_element_type=jnp.float32) + tb_ref[...]
    )
    nt2_ref[0] = nt[:, :64]
    nt2_ref[1] = nt[:, 64:]


def _sc_body(recv_hbm, send_hbm, t_hbm, nt_hbm, z_hbm, out_hbm,
             acc, sidx, ridx, sadj, ts, tr, rows, pbuf, vals, nin, nout,
             sem1, sem2, sem3):
    g = lax.axis_index("c")
    sid = lax.axis_index("s")
    lane = lax.iota(jnp.int32, 16)
    zvec = jnp.zeros((16,), jnp.float32)

    pltpu.sync_copy(z_hbm.at[pl.ds(0, NPT)], acc.at[pl.ds(sid * NPT, NPT)])

    @pl.when(sid == 0)
    def _zero_tail():
        pltpu.sync_copy(z_hbm.at[pl.ds(0, NTAIL)],
                        acc.at[pl.ds(NSUB * NPT, NTAIL)])
    for ii in range(CH // 16):
        rowv = ii * 16 + lane
        for k in range(AW - 68):
            plsc.store_scatter(vals, [rowv, jnp.full((16,), 68 + k, jnp.int32)], zvec)
    plsc.subcore_barrier()

    nchunks = (E // CH - sid + NSUB - 1) // NSUB

    def chunk_body(i, carry):
        base = (sid + i * NSUB) * CH
        pltpu.sync_copy(send_hbm.at[pl.ds(base, CH)], sidx)
        pltpu.sync_copy(recv_hbm.at[pl.ds(base, CH)], ridx)
        for ii in range(CH // 16):
            sl = pl.ds(ii * 16, 16)
            sadj[sl] = sidx[sl] + g * N
        cp1 = pltpu.async_copy(t_hbm.at[sidx], ts, sem1)
        cp2 = pltpu.async_copy(t_hbm.at[ridx], tr, sem2)
        cp3 = pltpu.async_copy(nt_hbm.at[sadj], rows, sem3)
        cp1.wait()
        cp2.wait()
        for ii in range(CH // 16):
            rowv = ii * 16 + lane
            for h in range(4):
                cs = jnp.full((16,), 0, jnp.int32) + (4 * g + h)
                cr = jnp.full((16,), 8, jnp.int32) + (4 * g + h)
                a_s = plsc.load_gather(ts, [rowv, cs])
                a_r = plsc.load_gather(tr, [rowv, cr])
                p = jnp.exp(a_s + a_r)
                plsc.store_scatter(pbuf, [rowv, jnp.full((16,), h, jnp.int32)], p)
                plsc.store_scatter(vals, [rowv, jnp.full((16,), 64 + h, jnp.int32)], p)
        cp3.wait()

        def edge_body(j, c2):
            for h in range(4):
                pj = plsc.load_gather(
                    pbuf, [jnp.full((16,), j, jnp.int32),
                           jnp.full((16,), h, jnp.int32)])
                vals[j, pl.ds(h * 16, 16)] = rows[j, pl.ds(h * 16, 16)] * pj
            return c2

        lax.fori_loop(0, CH, edge_body, 0)
        pltpu.sync_copy(vals, acc.at[ridx], add=True)
        return carry

    lax.fori_loop(0, nchunks, chunk_body, 0)
    plsc.subcore_barrier()

    def node_body(n, carry):
        denv = nin[n, pl.ds(56, 16)]
        for h in range(4):
            den = jnp.maximum(denv[8 + h], 1e-37)
            nout[n, pl.ds(h * 16, 16)] = nin[n, pl.ds(h * 16, 16)] / den
        return carry

    def norm_chunk(ci, carry):
        base = sid * NPT + ci * NCN
        pltpu.sync_copy(acc.at[pl.ds(base, NCN)], nin)
        lax.fori_loop(0, NCN, node_body, 0)
        pltpu.sync_copy(nout, out_hbm.at[g, pl.ds(base, NCN)])
        return carry

    lax.fori_loop(0, NPT // NCN, norm_chunk, 0)

    @pl.when(sid == 0)
    def _norm_tail():
        pltpu.sync_copy(acc.at[pl.ds(NSUB * NPT, NTAIL)], nin.at[pl.ds(0, NTAIL)])
        lax.fori_loop(0, NTAIL, node_body, 0)
        pltpu.sync_copy(nout.at[pl.ds(0, NTAIL)],
                        out_hbm.at[g, pl.ds(NSUB * NPT, NTAIL)])


_sc_kernel = functools.partial(
    pl.kernel,
    out_type=jax.ShapeDtypeStruct((2, N, 64), jnp.float32),
    mesh=plsc.VectorSubcoreMesh(core_axis_name="c", subcore_axis_name="s"),
    compiler_params=pltpu.CompilerParams(
        use_tc_tiling_on_sc=False, needs_layout_passes=False),
    scratch_types=[
        pltpu.VMEM_SHARED((N, AW), jnp.float32),
        pltpu.VMEM((CH,), jnp.int32),
        pltpu.VMEM((CH,), jnp.int32),
        pltpu.VMEM((CH,), jnp.int32),
        pltpu.VMEM((CH, 16), jnp.float32),
        pltpu.VMEM((CH, 16), jnp.float32),
        pltpu.VMEM((CH, 64), jnp.float32),
        pltpu.VMEM((CH, 4), jnp.float32),
        pltpu.VMEM((CH, AW), jnp.float32),
        pltpu.VMEM((NCN, AW), jnp.float32),
        pltpu.VMEM((NCN, 64), jnp.float32),
        pltpu.SemaphoreType.DMA,
        pltpu.SemaphoreType.DMA,
        pltpu.SemaphoreType.DMA,
    ],
)(_sc_body)


def kernel(nodes, receivers, senders, Wq, bq, Wl, bl):
    d = jnp.arange(DIM)
    oh = jax.nn.one_hot(d // 16, H, dtype=jnp.float32)
    wl = Wl[:, 0]
    sw = jnp.concatenate(
        [oh * wl[d % 16][:, None], oh * wl[16 + (d % 16)][:, None]], axis=1)
    tb = jnp.concatenate(
        [jnp.broadcast_to(bl, (H,)), jnp.zeros((H,), jnp.float32)]).reshape(1, 2 * H)

    nt2, t = pl.pallas_call(
        _tc_body,
        grid=(N // ROW_BLOCK,),
        in_specs=[
            pl.BlockSpec((ROW_BLOCK, DIM), lambda b: (b, 0)),
            pl.BlockSpec((DIM, DIM), lambda b: (0, 0)),
            pl.BlockSpec((1, DIM), lambda b: (0, 0)),
            pl.BlockSpec((DIM, 2 * H), lambda b: (0, 0)),
            pl.BlockSpec((1, 2 * H), lambda b: (0, 0)),
        ],
        out_specs=[
            pl.BlockSpec((2, ROW_BLOCK, 64), lambda b: (0, b, 0)),
            pl.BlockSpec((ROW_BLOCK, 2 * H), lambda b: (b, 0)),
        ],
        out_shape=[
            jax.ShapeDtypeStruct((2, N, 64), jnp.float32),
            jax.ShapeDtypeStruct((N, 2 * H), jnp.float32),
        ],
    )(nodes, Wq, bq.reshape(1, DIM), sw, tb)

    nt_flat = nt2.reshape(2 * N, 64)
    z = jnp.zeros((NPT, AW), jnp.float32)
    out2 = _sc_kernel(receivers, senders, t, nt_flat, z)
    return jnp.concatenate([out2[0], out2[1]], axis=1)
